# Initial kernel scaffold; baseline (speedup 1.0000x reference)
import jax
import jax.numpy as jnp
from jax.experimental import pallas as pl


def kernel(x, wih_0, whh_0, bih_0, bhh_0, wih_t_pad_0, whh_t_pad_0, b_fold_0, bhn_pad_0, wih_1, whh_1, bih_1, bhh_1, wih_t_pad_1, whh_t_pad_1, b_fold_1, bhn_pad_1):
    raise NotImplementedError("write your pallas kernel here")


# speedup vs baseline: 1.3080x; 1.3080x over previous
import functools

import jax
import jax.numpy as jnp
from jax.experimental import pallas as pl
from jax.experimental.pallas import tpu as pltpu


def _const_spec(block_shape, index_map):
    try:
        return pl.BlockSpec(block_shape, index_map,
                            pipeline_mode=pl.Buffered(1))
    except (AttributeError, TypeError):
        return pl.BlockSpec(block_shape, index_map)


def _gru2_kernel(x_ref, wcat1_ref, whn1_ref, win1_ref, brz1_ref, bn1_ref, bhn1_ref,
                 wcat2_ref, whn2_ref, win2_ref, brz2_ref, bn2_ref, bhn2_ref,
                 out_ref, hn_ref, h1_scr, h2_scr):
    t = pl.program_id(1)
    nt = pl.num_programs(1)

    @pl.when(t == 0)
    def _():
        h1_scr[...] = jnp.zeros_like(h1_scr)
        h2_scr[...] = jnp.zeros_like(h2_scr)

    Bb, Hp = h1_scr.shape
    Tc = x_ref.shape[0]

    def gru_step(h, xin_bf, wcat_ref, whn_ref, win_ref, brz_ref, bn_ref,
                 bhn_ref):
        hb = h.astype(jnp.bfloat16)
        a = jnp.concatenate([hb, xin_bf], axis=1)
        rz = jnp.dot(a, wcat_ref[...],
                     preferred_element_type=jnp.float32) + brz_ref[...]
        r = jax.nn.sigmoid(rz[:, :Hp])
        z = jax.nn.sigmoid(rz[:, Hp:])
        gh_n = jnp.dot(hb, whn_ref[...],
                       preferred_element_type=jnp.float32)
        gi_n = jnp.dot(xin_bf, win_ref[...],
                       preferred_element_type=jnp.float32) + bn_ref[...]
        n = jnp.tanh(gi_n + r * (gh_n + bhn_ref[...]))
        return (1.0 - z) * n + z * h

    def step1(h1, i):
        x_bf = x_ref[i].astype(jnp.bfloat16)
        h1n = gru_step(h1, x_bf, wcat1_ref, whn1_ref, win1_ref,
                       brz1_ref, bn1_ref, bhn1_ref)
        return h1n, h1n.astype(jnp.bfloat16)

    def step2(h2, o1_bf, i):
        h2n = gru_step(h2, o1_bf, wcat2_ref, whn2_ref, win2_ref,
                       brz2_ref, bn2_ref, bhn2_ref)
        out_ref[i] = h2n.astype(jnp.bfloat16).astype(jnp.float32)
        return h2n

    h1, o1 = step1(h1_scr[...], 0)

    def body(i, carry):
        h1, h2, o1 = carry
        h1n, o1n = step1(h1, i)
        h2n = step2(h2, o1, i - 1)
        return h1n, h2n, o1n

    h1, h2, o1 = jax.lax.fori_loop(1, Tc, body, (h1, h2_scr[...], o1),
                                   unroll=4)
    h2 = step2(h2, o1, Tc - 1)

    h1_scr[...] = h1
    h2_scr[...] = h2

    @pl.when(t == nt - 1)
    def _():
        hn_ref[0] = h1.astype(jnp.bfloat16).astype(jnp.float32)
        hn_ref[1] = h2.astype(jnp.bfloat16).astype(jnp.float32)


@functools.partial(jax.jit, static_argnames=("time_chunk", "batch_block"))
def _fused_forward(x, wih1, bf1, whh1, bhn1, wih2, bf2, whh2, bhn2,
                   *, time_chunk=16, batch_block=128):
    T, B, In = x.shape
    Hp = whh1.shape[0]
    Bp = ((B + batch_block - 1) // batch_block) * batch_block
    if Bp != B:
        x = jnp.pad(x, ((0, 0), (0, Bp - B), (0, 0)))
    nb = Bp // batch_block
    nt = T // time_chunk
    H2 = 2 * Hp

    wcat1 = jnp.concatenate([whh1[:, :H2], wih1[:, :H2]], axis=0)
    wcat2 = jnp.concatenate([whh2[:, :H2], wih2[:, :H2]], axis=0)
    whn1, win1 = whh1[:, H2:], wih1[:, H2:]
    whn2, win2 = whh2[:, H2:], wih2[:, H2:]
    brz1, bn1 = bf1[:, :H2], bf1[:, H2:]
    brz2, bn2 = bf2[:, :H2], bf2[:, H2:]

    out, hn = pl.pallas_call(
        _gru2_kernel,
        out_shape=(jax.ShapeDtypeStruct((T, Bp, Hp), jnp.float32),
                   jax.ShapeDtypeStruct((2, Bp, Hp), jnp.float32)),
        grid=(nb, nt),
        in_specs=[
            pl.BlockSpec((time_chunk, batch_block, In),
                         lambda b, t: (t, b, 0)),
            _const_spec((H2, H2), lambda b, t: (0, 0)),
            _const_spec((Hp, Hp), lambda b, t: (0, 0)),
            _const_spec((Hp, Hp), lambda b, t: (0, 0)),
            _const_spec((1, H2), lambda b, t: (0, 0)),
            _const_spec((1, Hp), lambda b, t: (0, 0)),
            _const_spec((1, Hp), lambda b, t: (0, 0)),
            _const_spec((H2, H2), lambda b, t: (0, 0)),
            _const_spec((Hp, Hp), lambda b, t: (0, 0)),
            _const_spec((Hp, Hp), lambda b, t: (0, 0)),
            _const_spec((1, H2), lambda b, t: (0, 0)),
            _const_spec((1, Hp), lambda b, t: (0, 0)),
            _const_spec((1, Hp), lambda b, t: (0, 0)),
        ],
        out_specs=(
            pl.BlockSpec((time_chunk, batch_block, Hp),
                         lambda b, t: (t, b, 0)),
            pl.BlockSpec((2, batch_block, Hp), lambda b, t: (0, b, 0)),
        ),
        scratch_shapes=[
            pltpu.VMEM((batch_block, Hp), jnp.float32),
            pltpu.VMEM((batch_block, Hp), jnp.float32),
        ],
        compiler_params=pltpu.CompilerParams(
            dimension_semantics=("parallel", "arbitrary"),
            vmem_limit_bytes=48 * 1024 * 1024),
    )(x, wcat1, whn1, win1, brz1, bn1, bhn1,
      wcat2, whn2, win2, brz2, bn2, bhn2)

    return out[:, :B, :], hn[:, :B, :]


def kernel(x, wih_0, whh_0, bih_0, bhh_0, wih_t_pad_0, whh_t_pad_0,
           b_fold_0, bhn_pad_0,
           wih_1, whh_1, bih_1, bhh_1, wih_t_pad_1, whh_t_pad_1,
           b_fold_1, bhn_pad_1):
    return _fused_forward(
        x,
        wih_t_pad_0, b_fold_0.reshape(1, -1), whh_t_pad_0, bhn_pad_0,
        wih_t_pad_1, b_fold_1.reshape(1, -1), whh_t_pad_1, bhn_pad_1)
